# Initial kernel scaffold; baseline (speedup 1.0000x reference)
import jax
import jax.numpy as jnp
from jax.experimental import pallas as pl


def kernel(input_):
    raise NotImplementedError("write your pallas kernel here")


# speedup vs baseline: 218.8747x; 218.8747x over previous
import functools

import jax
import jax.numpy as jnp
from jax import lax
from jax.experimental import pallas as pl
from jax.experimental.pallas import tpu as pltpu
from jax.experimental.pallas import tpu_sc as plsc

B = 8
L = 4096
MIN_DIST = 32
NLANES = 16
NCHUNKS = L // NLANES
NBLOCKS = 16
CHUNKS_PER_BLOCK = NCHUNKS // NBLOCKS
BLOCK = L // NBLOCKS
NEG = float("-inf")


def _nms_body(x_hbm, out_hbm, x_v, vals_v, out_v):
    w = lax.axis_index("s") * 2 + lax.axis_index("c")

    @pl.when(w < B)
    def _():
        b = w
        pltpu.sync_copy(x_hbm.at[b], x_v)

        lane = lax.broadcasted_iota(jnp.int32, (NLANES,), 0)
        zeros = jnp.zeros((NLANES,), jnp.float32)
        ninf = jnp.full((NLANES,), NEG, jnp.float32)

        def block_pass(j, bmax_vec):
            def chunk_pass(c, acc):
                idx = (j * CHUNKS_PER_BLOCK + c) * NLANES + lane
                xc = plsc.load_gather(x_v, [idx])
                xm = plsc.load_gather(x_v, [jnp.maximum(idx - 1, 0)])
                xr = plsc.load_gather(x_v, [jnp.minimum(idx + 1, L - 1)])
                dl = xc - xm
                dr = xr - xc
                pos = xc > 0.0
                peak = pos & (dr <= 0.0) & (dl > 0.0)
                valley = (~pos) & (dr > 0.0) & (dl <= 0.0)
                v = jnp.where(peak | valley, jnp.abs(xc), NEG)
                plsc.store_scatter(vals_v, [idx], v)
                plsc.store_scatter(out_v, [idx], zeros)
                return jnp.maximum(acc, v)

            acc = lax.fori_loop(0, CHUNKS_PER_BLOCK, chunk_pass, ninf)
            return jnp.where(lane == j, jnp.max(acc), bmax_vec)

        bmax_vec = lax.fori_loop(0, NBLOCKS, block_pass, ninf)

        def block_max(jj):
            def scan(c, acc):
                idx = jj * BLOCK + c * NLANES + lane
                return jnp.maximum(acc, plsc.load_gather(vals_v, [idx]))

            return jnp.max(lax.fori_loop(0, CHUNKS_PER_BLOCK, scan, ninf))

        def greedy_cond(carry):
            m, _ = carry
            return m > NEG

        def greedy_body(carry):
            m, bmax_vec = carry
            j = jnp.min(jnp.where(bmax_vec == m, lane, NBLOCKS))

            def scan(c, carry):
                cur_val, cur_idx = carry
                idx = j * BLOCK + c * NLANES + lane
                v = plsc.load_gather(vals_v, [idx])
                upd = v > cur_val
                return jnp.where(upd, v, cur_val), jnp.where(upd, idx, cur_idx)

            cur_val, cur_idx = lax.fori_loop(
                0, CHUNKS_PER_BLOCK, scan, (ninf, jnp.full((NLANES,), L, jnp.int32))
            )
            p = jnp.min(jnp.where(cur_val == m, cur_idx, jnp.int32(1 << 30)))

            pidx = jnp.full((NLANES,), p, jnp.int32)
            xp = plsc.load_gather(x_v, [pidx])
            plsc.store_scatter(out_v, [pidx], xp, mask=lane == 0)

            lo = jnp.maximum(p - MIN_DIST, 0)
            hi = jnp.minimum(p + MIN_DIST, L - 1)
            for k in range((2 * MIN_DIST) // NLANES + 1):
                sidx = lo + k * NLANES + lane
                plsc.store_scatter(
                    vals_v, [jnp.minimum(sidx, L - 1)], ninf, mask=sidx <= hi
                )

            jlo = lo // BLOCK
            jhi = hi // BLOCK
            bmax_vec = jnp.where(lane == jlo, block_max(jlo), bmax_vec)
            bmax_vec = jnp.where(lane == jhi, block_max(jhi), bmax_vec)
            return jnp.max(bmax_vec), bmax_vec

        m0 = jnp.max(bmax_vec)
        lax.while_loop(greedy_cond, greedy_body, (m0, bmax_vec))

        pltpu.sync_copy(out_v, out_hbm.at[b])


@jax.jit
def _nms(x):
    run = pl.kernel(
        _nms_body,
        out_type=jax.ShapeDtypeStruct((B, L), jnp.float32),
        mesh=plsc.VectorSubcoreMesh(core_axis_name="c", subcore_axis_name="s"),
        compiler_params=pltpu.CompilerParams(needs_layout_passes=False),
        scratch_types=[
            pltpu.VMEM((L,), jnp.float32),
            pltpu.VMEM((L,), jnp.float32),
            pltpu.VMEM((L,), jnp.float32),
        ],
    )
    return run(x)


def kernel(input_):
    return _nms(input_.reshape(B, L)).reshape(B, 1, L)


# speedup vs baseline: 303.1711x; 1.3851x over previous
import functools

import jax
import jax.numpy as jnp
from jax import lax
from jax.experimental import pallas as pl
from jax.experimental.pallas import tpu as pltpu
from jax.experimental.pallas import tpu_sc as plsc

B = 8
L = 4096
MIN_DIST = 32
NLANES = 16
NBLOCKS = 32
BLOCK = L // NBLOCKS
CHUNKS_PER_BLOCK = BLOCK // NLANES
NEG = float("-inf")
BIGI = 1 << 30


def _nms_body(x_hbm, out_hbm, x_v, vals_v, out_v):
    w = lax.axis_index("s") * 2 + lax.axis_index("c")

    @pl.when(w < B)
    def _():
        b = w
        pltpu.sync_copy(x_hbm.at[b], x_v)

        lane = lax.broadcasted_iota(jnp.int32, (NLANES,), 0)
        zeros = jnp.zeros((NLANES,), jnp.float32)
        ninf = jnp.full((NLANES,), NEG, jnp.float32)

        def bupdate(jj, bm, b0, b1):
            sel = lane == (jj & (NLANES - 1))
            lo_half = jj < NLANES
            b0 = jnp.where(sel & lo_half, bm, b0)
            b1 = jnp.where(sel & (~lo_half), bm, b1)
            return b0, b1

        def block_pass(j, carry):
            b0, b1 = carry

            def chunk_pass(c, acc):
                idx = j * BLOCK + c * NLANES + lane
                xc = plsc.load_gather(x_v, [idx])
                xm = plsc.load_gather(x_v, [jnp.maximum(idx - 1, 0)])
                xr = plsc.load_gather(x_v, [jnp.minimum(idx + 1, L - 1)])
                dl = xc - xm
                dr = xr - xc
                pos = xc > 0.0
                peak = pos & (dr <= 0.0) & (dl > 0.0)
                valley = (~pos) & (dr > 0.0) & (dl <= 0.0)
                v = jnp.where(peak | valley, jnp.abs(xc), NEG)
                plsc.store_scatter(vals_v, [idx], v)
                plsc.store_scatter(out_v, [idx], zeros)
                return jnp.maximum(acc, v)

            acc = lax.fori_loop(0, CHUNKS_PER_BLOCK, chunk_pass, ninf)
            return bupdate(j, jnp.max(acc), b0, b1)

        b0, b1 = lax.fori_loop(0, NBLOCKS, block_pass, (ninf, ninf))

        def block_max(jj):
            def scan(c, acc):
                idx = jj * BLOCK + c * NLANES + lane
                return jnp.maximum(acc, plsc.load_gather(vals_v, [idx]))

            return jnp.max(lax.fori_loop(0, CHUNKS_PER_BLOCK, scan, ninf))

        def greedy_cond(carry):
            m, _, _ = carry
            return m > NEG

        def greedy_body(carry):
            m, b0, b1 = carry
            w0 = jnp.where(b0 == m, lane, BIGI)
            w1 = jnp.where(b1 == m, lane + NLANES, BIGI)
            j = jnp.min(jnp.minimum(w0, w1))

            def scan(c, carry):
                cur_val, cur_idx = carry
                idx = j * BLOCK + c * NLANES + lane
                v = plsc.load_gather(vals_v, [idx])
                upd = v > cur_val
                return jnp.where(upd, v, cur_val), jnp.where(upd, idx, cur_idx)

            cur_val, cur_idx = lax.fori_loop(
                0, CHUNKS_PER_BLOCK, scan,
                (ninf, jnp.full((NLANES,), L, jnp.int32)),
            )
            p = jnp.min(jnp.where(cur_val == m, cur_idx, BIGI))

            pidx = jnp.full((NLANES,), p, jnp.int32)
            xp = plsc.load_gather(x_v, [pidx])
            plsc.store_scatter(out_v, [pidx], xp, mask=lane == 0)

            lo = jnp.maximum(p - MIN_DIST, 0)
            hi = jnp.minimum(p + MIN_DIST, L - 1)
            for k in range((2 * MIN_DIST) // NLANES + 1):
                sidx = lo + k * NLANES + lane
                plsc.store_scatter(
                    vals_v, [jnp.minimum(sidx, L - 1)], ninf, mask=sidx <= hi
                )

            jlo = lo // BLOCK
            jhi = hi // BLOCK
            b0, b1 = bupdate(jlo, block_max(jlo), b0, b1)
            b0, b1 = lax.cond(
                jhi != jlo,
                lambda b0, b1: bupdate(jhi, block_max(jhi), b0, b1),
                lambda b0, b1: (b0, b1),
                b0, b1,
            )
            return jnp.max(jnp.maximum(b0, b1)), b0, b1

        m0 = jnp.max(jnp.maximum(b0, b1))
        lax.while_loop(greedy_cond, greedy_body, (m0, b0, b1))

        pltpu.sync_copy(out_v, out_hbm.at[b])


@jax.jit
def _nms(x):
    run = pl.kernel(
        _nms_body,
        out_type=jax.ShapeDtypeStruct((B, L), jnp.float32),
        mesh=plsc.VectorSubcoreMesh(core_axis_name="c", subcore_axis_name="s"),
        compiler_params=pltpu.CompilerParams(needs_layout_passes=False),
        scratch_types=[
            pltpu.VMEM((L,), jnp.float32),
            pltpu.VMEM((L,), jnp.float32),
            pltpu.VMEM((L,), jnp.float32),
        ],
    )
    return run(x)


def kernel(input_):
    return _nms(input_.reshape(B, L)).reshape(B, 1, L)
